# baseline (speedup 1.0000x reference)
import functools

import jax
import jax.numpy as jnp
from jax import lax
from jax.experimental import pallas as pl
from jax.experimental.pallas import tpu as pltpu
from jax.experimental.pallas import tpu_sc as plsc

_CHUNK = 128


@functools.lru_cache(maxsize=None)
def _make_kernel(B, D, VU, VJ, VS, VC):
    info = plsc.get_sparse_core_info()
    NC, NS, L = info.num_cores, info.num_subcores, info.num_lanes
    NW = NC * NS
    assert B % (NW * _CHUNK) == 0
    bpw = B // NW
    nc = bpw // _CHUNK
    nrow = D * nc
    mesh = plsc.VectorSubcoreMesh(core_axis_name="c", subcore_axis_name="s")

    @functools.partial(
        pl.kernel,
        mesh=mesh,
        out_type=jax.ShapeDtypeStruct((4 * D, B), jnp.float32),
        scratch_types=[
            pltpu.VMEM((nrow, _CHUNK), jnp.int32),
            pltpu.VMEM((nrow, _CHUNK), jnp.int32),
            pltpu.VMEM((bpw,), jnp.int32),
            pltpu.VMEM((bpw,), jnp.int32),
            pltpu.VMEM((D, VS), jnp.float32),
            pltpu.VMEM((D, VC), jnp.float32),
            pltpu.VMEM((4 * D, bpw), jnp.float32),
            pltpu.SemaphoreType.DMA,
        ],
        compiler_params=pltpu.CompilerParams(needs_layout_passes=False),
    )
    def k(idxu_hbm, idxj_hbm, size_hbm, country_hbm,
          ut, jt, st, ct, out_hbm,
          iu, ij, isz, ic, stv, ctv, stage, sem):
        wid = lax.axis_index("s") * NC + lax.axis_index("c")
        base = wid * bpw
        pltpu.sync_copy(idxu_hbm.at[pl.ds(wid * nrow, nrow)], iu)
        pltpu.sync_copy(idxj_hbm.at[pl.ds(wid * nrow, nrow)], ij)
        pltpu.sync_copy(size_hbm.at[pl.ds(base, bpw)], isz)
        pltpu.sync_copy(country_hbm.at[pl.ds(base, bpw)], ic)
        pltpu.sync_copy(st, stv)
        pltpu.sync_copy(ct, ctv)

        copies = []
        for d in range(D):
            for c in range(nc):
                sl = pl.ds(c * _CHUNK, _CHUNK)
                copies.append(pltpu.async_copy(
                    ut.at[iu.at[d * nc + c]], stage.at[d, sl], sem))
                copies.append(pltpu.async_copy(
                    jt.at[ij.at[d * nc + c]], stage.at[D + d, sl], sem))

        iota = lax.iota(jnp.int32, L)

        def small(g, carry):
            sl = pl.ds(g * L, L)
            svec = isz[sl]
            cvec = ic[sl]
            for d in range(D):
                dvec = jnp.full((L,), d, jnp.int32)
                stage[2 * D + d, sl] = plsc.load_gather(stv, [dvec, svec])
                stage[3 * D + d, sl] = plsc.load_gather(ctv, [dvec, cvec])
            return carry

        lax.fori_loop(0, bpw // L, small, 0)

        for c in copies:
            c.wait()
        pltpu.sync_copy(stage, out_hbm.at[:, pl.ds(base, bpw)])

    return k


def kernel(email_address, job_title, company_size, country,
           user_table, job_table, size_table, country_table):
    B = email_address.shape[0]
    VU, D = user_table.shape
    VJ = job_table.shape[0]
    VS = size_table.shape[0]
    VC = country_table.shape[0]
    k = _make_kernel(B, D, VU, VJ, VS, VC)
    bpw = B // 32
    nc = bpw // _CHUNK
    dofs = jnp.arange(D, dtype=jnp.int32)
    e3 = email_address.reshape(32, nc, _CHUNK)
    j3 = job_title.reshape(32, nc, _CHUNK)
    idxu = (e3[:, None] + (dofs * VU)[None, :, None, None]).reshape(-1, _CHUNK)
    idxj = (j3[:, None] + (dofs * VJ)[None, :, None, None]).reshape(-1, _CHUNK)
    out_t = k(idxu, idxj, company_size, country,
              user_table.T.reshape(D * VU), job_table.T.reshape(D * VJ),
              size_table.T, country_table.T)
    return out_t.T


# speedup vs baseline: 4.1451x; 4.1451x over previous
import functools

import jax
import jax.numpy as jnp
from jax import lax
from jax.experimental import pallas as pl
from jax.experimental.pallas import tpu as pltpu
from jax.experimental.pallas import tpu_sc as plsc


@functools.lru_cache(maxsize=None)
def _make_kernel(B, D, VU, VJ, VS, VC):
    info = plsc.get_sparse_core_info()
    NC, NS, L = info.num_cores, info.num_subcores, info.num_lanes
    NW = NC * NS
    G = 128 // D
    assert D * G == 128 and VU % G == 0 and VJ % G == 0
    assert B % NW == 0
    bpw = B // NW
    cs = min(bpw, 128)
    assert bpw % cs == 0 and cs % L == 0
    mesh = plsc.VectorSubcoreMesh(core_axis_name="c", subcore_axis_name="s")

    @functools.partial(
        pl.kernel,
        mesh=mesh,
        out_type=jax.ShapeDtypeStruct((B, 4 * D), jnp.float32),
        scratch_types=[
            pltpu.VMEM((cs,), jnp.int32),
            pltpu.VMEM((cs,), jnp.int32),
            pltpu.VMEM((cs,), jnp.int32),
            pltpu.VMEM((cs,), jnp.int32),
            pltpu.VMEM((cs,), jnp.int32),
            pltpu.VMEM((cs,), jnp.int32),
            pltpu.VMEM((cs, G * D), jnp.float32),
            pltpu.VMEM((cs, G * D), jnp.float32),
            pltpu.VMEM((VS, D), jnp.float32),
            pltpu.VMEM((VC, D), jnp.float32),
            pltpu.VMEM((cs, 4 * D), jnp.float32),
            pltpu.SemaphoreType.DMA,
        ],
        compiler_params=pltpu.CompilerParams(needs_layout_passes=False),
    )
    def k(email_hbm, job_hbm, size_hbm, country_hbm,
          user_t, job_t, size_t, country_t, out_hbm,
          iu, ij, isz, ic, gu, gj, ru, rj, ts, tc, stage, sem):
        wid = lax.axis_index("s") * NC + lax.axis_index("c")
        pltpu.sync_copy(size_t, ts)
        pltpu.sync_copy(country_t, tc)
        def chunk_body(ch, carry):
            base = wid * bpw + ch * cs
            pltpu.sync_copy(email_hbm.at[pl.ds(base, cs)], iu)
            pltpu.sync_copy(job_hbm.at[pl.ds(base, cs)], ij)
            pltpu.sync_copy(size_hbm.at[pl.ds(base, cs)], isz)
            pltpu.sync_copy(country_hbm.at[pl.ds(base, cs)], ic)
            for g in range(cs // L):
                sl = pl.ds(g * L, L)
                gu[sl] = iu[sl] >> 2
                gj[sl] = ij[sl] >> 2
            cu = pltpu.async_copy(user_t.at[gu], ru, sem)
            cj = pltpu.async_copy(job_t.at[gj], rj, sem)
            iota = lax.iota(jnp.int32, L)
            cu.wait()
            cj.wait()
            for g in range(cs // L):
                sl = pl.ds(g * L, L)
                jvec = g * L + iota
                ucol = (iu[sl] & (G - 1)) * D
                jcol = (ij[sl] & (G - 1)) * D
                srow = isz[sl]
                crow = ic[sl]
                for c in range(D):
                    cvec = jnp.full((L,), c, jnp.int32)
                    plsc.store_scatter(
                        stage, [jvec, cvec],
                        plsc.load_gather(ru, [jvec, ucol + c]))
                    plsc.store_scatter(
                        stage, [jvec, cvec + D],
                        plsc.load_gather(rj, [jvec, jcol + c]))
                    plsc.store_scatter(
                        stage, [jvec, cvec + 2 * D],
                        plsc.load_gather(ts, [srow, cvec]))
                    plsc.store_scatter(
                        stage, [jvec, cvec + 3 * D],
                        plsc.load_gather(tc, [crow, cvec]))
            pltpu.sync_copy(stage, out_hbm.at[pl.ds(base, cs)])
            return carry

        lax.fori_loop(0, bpw // cs, chunk_body, 0)

    return k


def kernel(email_address, job_title, company_size, country,
           user_table, job_table, size_table, country_table):
    B = email_address.shape[0]
    VU, D = user_table.shape
    VJ = job_table.shape[0]
    VS = size_table.shape[0]
    VC = country_table.shape[0]
    k = _make_kernel(B, D, VU, VJ, VS, VC)
    return k(email_address, job_title, company_size, country,
             user_table.reshape(VU // 4, 4 * D),
             job_table.reshape(VJ // 4, 4 * D),
             size_table, country_table)


# speedup vs baseline: 4.2297x; 1.0204x over previous
import functools

import jax
import jax.numpy as jnp
from jax import lax
from jax.experimental import pallas as pl
from jax.experimental.pallas import tpu as pltpu
from jax.experimental.pallas import tpu_sc as plsc

_CHUNK = 128


@functools.lru_cache(maxsize=None)
def _make_kernel(B, D, VU, VJ, VS, VC):
    info = plsc.get_sparse_core_info()
    NC, NS, L = info.num_cores, info.num_subcores, info.num_lanes
    NW = NC * NS
    G = 128 // D
    assert D * G == 128 and VU % G == 0 and VJ % G == 0
    assert B % (NW * _CHUNK) == 0
    bpw = B // NW
    nch = bpw // _CHUNK
    mesh = plsc.VectorSubcoreMesh(core_axis_name="c", subcore_axis_name="s")

    res_t = pltpu.VMEM((_CHUNK, G * D), jnp.float32)

    @functools.partial(
        pl.kernel,
        mesh=mesh,
        out_type=jax.ShapeDtypeStruct((B, 4 * D), jnp.float32),
        scratch_types=[
            pltpu.VMEM((bpw,), jnp.int32),
            pltpu.VMEM((bpw,), jnp.int32),
            pltpu.VMEM((bpw,), jnp.int32),
            pltpu.VMEM((bpw,), jnp.int32),
            pltpu.VMEM((nch, _CHUNK), jnp.int32),
            pltpu.VMEM((nch, _CHUNK), jnp.int32),
            res_t, res_t,
            res_t, res_t,
            pltpu.VMEM((VS, D), jnp.float32),
            pltpu.VMEM((VC, D), jnp.float32),
            pltpu.VMEM((_CHUNK, 4 * D), jnp.float32),
            pltpu.SemaphoreType.DMA,
            pltpu.SemaphoreType.DMA,
        ],
        compiler_params=pltpu.CompilerParams(needs_layout_passes=False),
    )
    def k(email_hbm, job_hbm, size_hbm, country_hbm,
          user_t, job_t, size_t, country_t, out_hbm,
          iu, ij, isz, ic, gu, gj, ru0, ru1, rj0, rj1, ts, tc, stage,
          su, sj):
        wid = lax.axis_index("s") * NC + lax.axis_index("c")
        base = wid * bpw
        pltpu.sync_copy(email_hbm.at[pl.ds(base, bpw)], iu)
        pltpu.sync_copy(job_hbm.at[pl.ds(base, bpw)], ij)
        pltpu.sync_copy(size_hbm.at[pl.ds(base, bpw)], isz)
        pltpu.sync_copy(country_hbm.at[pl.ds(base, bpw)], ic)
        pltpu.sync_copy(size_t, ts)
        pltpu.sync_copy(country_t, tc)

        for j in range(nch):
            def gran(g2, carry, j=j):
                isl = pl.ds(j * _CHUNK + g2 * L, L)
                osl = pl.ds(g2 * L, L)
                gu[j, osl] = iu[isl] >> 2
                gj[j, osl] = ij[isl] >> 2
                return carry

            lax.fori_loop(0, _CHUNK // L, gran, 0)

        rus = [ru0, ru1]
        rjs = [rj0, rj1]
        iota = lax.iota(jnp.int32, L)

        def fire(j):
            return (pltpu.async_copy(user_t.at[gu.at[j]], rus[j % 2], su),
                    pltpu.async_copy(job_t.at[gj.at[j]], rjs[j % 2], sj))

        inflight = fire(0)
        for j in range(nch):
            cu, cj = inflight
            if j + 1 < nch:
                nxt = fire(j + 1)
            cu.wait()
            cj.wait()
            if j + 1 < nch:
                inflight = nxt
            ru = rus[j % 2]
            rj = rjs[j % 2]

            def body(g, carry, j=j, ru=ru, rj=rj):
                isl = pl.ds(j * _CHUNK + g * L, L)
                rvec = g * L + iota
                ucol = (iu[isl] & (G - 1)) * D
                jcol = (ij[isl] & (G - 1)) * D
                svec = isz[isl]
                cvec = ic[isl]
                for c in range(D):
                    cv = jnp.full((L,), c, jnp.int32)
                    plsc.store_scatter(
                        stage, [rvec, cv],
                        plsc.load_gather(ru, [rvec, ucol + c]))
                    plsc.store_scatter(
                        stage, [rvec, cv + D],
                        plsc.load_gather(rj, [rvec, jcol + c]))
                    plsc.store_scatter(
                        stage, [rvec, cv + 2 * D],
                        plsc.load_gather(ts, [svec, cv]))
                    plsc.store_scatter(
                        stage, [rvec, cv + 3 * D],
                        plsc.load_gather(tc, [cvec, cv]))
                return carry

            lax.fori_loop(0, _CHUNK // L, body, 0)
            pltpu.sync_copy(
                stage, out_hbm.at[pl.ds(base + j * _CHUNK, _CHUNK)])

    return k


def kernel(email_address, job_title, company_size, country,
           user_table, job_table, size_table, country_table):
    B = email_address.shape[0]
    VU, D = user_table.shape
    VJ = job_table.shape[0]
    VS = size_table.shape[0]
    VC = country_table.shape[0]
    k = _make_kernel(B, D, VU, VJ, VS, VC)
    return k(email_address, job_title, company_size, country,
             user_table.reshape(VU // 4, 4 * D),
             job_table.reshape(VJ // 4, 4 * D),
             size_table, country_table)
